# Initial kernel scaffold; baseline (speedup 1.0000x reference)
import functools

import jax
import jax.numpy as jnp
from jax import lax
from jax.experimental import pallas as pl
from jax.experimental.pallas import tpu as pltpu
from jax.experimental.pallas import tpu_sc as plsc

B = 4096
L = 128
D = 64
N = B * L
EPS = 1e-5

NUM_WORKERS = 32
ROWS_PER_W = N // NUM_WORKERS
CHUNK = 512
SUB = CHUNK // 128
N_CHUNKS = ROWS_PER_W // CHUNK


def _rsqrt(x):
    i = lax.bitcast_convert_type(x, jnp.int32)
    i = jnp.int32(0x5F3759DF) - lax.shift_right_arithmetic(i, 1)
    y = lax.bitcast_convert_type(i, jnp.float32)
    for _ in range(3):
        y = y * (1.5 - 0.5 * x * y * y)
    return y


def _body(x_ref, seg_ref, tok_ref, pos_ref, segt_ref, gam_ref, bet_ref,
          out_ref, idx_v, seg_v, rows_v, fused_v, pos_v, segt_v, gam_v,
          bet_v, sem):
    wid = lax.axis_index("s") * 2 + lax.axis_index("c")
    base_row = wid * ROWS_PER_W

    pltpu.sync_copy(pos_ref, pos_v)
    pltpu.sync_copy(segt_ref, segt_v)
    pltpu.sync_copy(gam_ref, gam_v)
    pltpu.sync_copy(bet_ref, bet_v)

    def build(l, carry):
        for c in range(4):
            sl = pl.ds(c * 16, 16)
            p = pos_v[l, sl]
            fused_v[2 * l, sl] = p + segt_v[0, sl]
            fused_v[2 * l + 1, sl] = p + segt_v[1, sl]
        return carry
    lax.fori_loop(0, 128, build, 0)

    def do_chunk(k, carry):
        chunk_base = base_row + k * CHUNK
        r0 = chunk_base // 128
        pltpu.sync_copy(x_ref.at[pl.ds(r0, SUB)], idx_v)
        pltpu.sync_copy(seg_ref.at[pl.ds(r0, SUB)], seg_v)
        copies = [
            pltpu.async_copy(tok_ref.at[idx_v.at[j]],
                             rows_v.at[pl.ds(j * 128, 128)], sem)
            for j in range(SUB)
        ]
        for cpy in copies:
            cpy.wait()

        def do_pos(i, c2):
            for j in range(SUB):
                row = j * 128 + i
                s = seg_v[j, i]
                fi = 2 * i + s
                v = [rows_v[row, pl.ds(c * 16, 16)] +
                     fused_v[fi, pl.ds(c * 16, 16)] for c in range(4)]
                t = (v[0] + v[1]) + (v[2] + v[3])
                q = ((v[0] * v[0] + v[1] * v[1]) +
                     (v[2] * v[2] + v[3] * v[3]))
                tot = jnp.sum(t)
                sq = jnp.sum(q)
                mean = tot * (1.0 / 64.0)
                var = sq * (1.0 / 64.0) - mean * mean
                inv = _rsqrt(var + EPS)
                mb = lax.broadcast_in_dim(mean, (16,), ())
                ib = lax.broadcast_in_dim(inv, (16,), ())
                for c in range(4):
                    sl = pl.ds(c * 16, 16)
                    n = (v[c] - mb) * ib
                    rows_v[row, sl] = n * gam_v[sl] + bet_v[sl]
            return c2
        lax.fori_loop(0, 128, do_pos, 0)
        pltpu.sync_copy(rows_v, out_ref.at[pl.ds(chunk_base, CHUNK)])
        return carry
    lax.fori_loop(0, N_CHUNKS, do_chunk, 0)


@jax.jit
def _emb(x, seg, tok_table, pos_table, seg_table, gamma, beta):
    mesh = plsc.VectorSubcoreMesh(core_axis_name="c", subcore_axis_name="s")
    f = pl.kernel(
        _body,
        out_type=jax.ShapeDtypeStruct((N, D), jnp.float32),
        mesh=mesh,
        scratch_types=[
            pltpu.VMEM((SUB, 128), jnp.int32),
            pltpu.VMEM((SUB, 128), jnp.int32),
            pltpu.VMEM((CHUNK, D), jnp.float32),
            pltpu.VMEM((256, D), jnp.float32),
            pltpu.VMEM((128, D), jnp.float32),
            pltpu.VMEM((2, D), jnp.float32),
            pltpu.VMEM((D,), jnp.float32),
            pltpu.VMEM((D,), jnp.float32),
            pltpu.SemaphoreType.DMA,
        ],
    )
    return f(x, seg, tok_table, pos_table, seg_table, gamma, beta)


def kernel(x, seg, tok_table, pos_table, seg_table, gamma, beta):
    out = _emb(x, seg, tok_table, pos_table, seg_table, gamma, beta)
    return out.reshape(B, L, D)


# speedup vs baseline: 1.1515x; 1.1515x over previous
import jax
import jax.numpy as jnp
from jax import lax
from jax.experimental import pallas as pl
from jax.experimental.pallas import tpu as pltpu
from jax.experimental.pallas import tpu_sc as plsc

B = 4096
L = 128
D = 64
N = B * L
EPS = 1e-5

NUM_WORKERS = 32
ROWS_PER_W = N // NUM_WORKERS
CHUNK = 512
SUB = CHUNK // 128
N_CHUNKS = ROWS_PER_W // CHUNK

def _rsqrt(x):
    i = lax.bitcast_convert_type(x, jnp.int32)
    i = jnp.int32(0x5F3759DF) - lax.shift_right_arithmetic(i, 1)
    y = lax.bitcast_convert_type(i, jnp.float32)
    for _ in range(3):
        y = y * (1.5 - 0.5 * x * y * y)
    return y


def _body(x_ref, seg_ref, tok_ref, pos_ref, segt_ref, gam_ref, bet_ref,
          out_ref, idx_v, seg_v, rows_v, fused_v, pos_v, segt_v, gam_v,
          bet_v, sem):
    wid = lax.axis_index("s") * 2 + lax.axis_index("c")
    base_row = wid * ROWS_PER_W

    pltpu.sync_copy(pos_ref, pos_v)
    pltpu.sync_copy(segt_ref, segt_v)
    pltpu.sync_copy(gam_ref, gam_v)
    pltpu.sync_copy(bet_ref, bet_v)

    def build(l, carry):
        for c in range(4):
            sl = pl.ds(c * 16, 16)
            p = pos_v[l, sl]
            fused_v[2 * l, sl] = p + segt_v[0, sl]
            fused_v[2 * l + 1, sl] = p + segt_v[1, sl]
        return carry
    lax.fori_loop(0, 128, build, 0)

    iota2 = lax.shift_left(lax.broadcasted_iota(jnp.int32, (16,), 0), 1)

    def do_chunk(k, carry):
        chunk_base = base_row + k * CHUNK
        pltpu.sync_copy(x_ref.at[pl.ds(chunk_base, CHUNK)], idx_v)
        pltpu.sync_copy(seg_ref.at[pl.ds(chunk_base, CHUNK)], seg_v)
        copies = [
            pltpu.async_copy(tok_ref.at[idx_v.at[pl.ds(j * 128, 128)]],
                             rows_v.at[pl.ds(j * 128, 128)], sem)
            for j in range(SUB)
        ]
        for cpy in copies:
            cpy.wait()

        def do_grp(gg, c2):
            r0 = gg * 16
            lbase = (gg % 8) * 32
            seg16 = seg_v[pl.ds(r0, 16)]
            fi16 = (iota2 + seg16) + lax.broadcast_in_dim(lbase, (16,), ())
            for e in range(16):
                row = r0 + e
                fi = fi16[e]
                v = [rows_v[row, pl.ds(c * 16, 16)] +
                     fused_v[fi, pl.ds(c * 16, 16)] for c in range(4)]
                t = (v[0] + v[1]) + (v[2] + v[3])
                q = ((v[0] * v[0] + v[1] * v[1]) +
                     (v[2] * v[2] + v[3] * v[3]))
                tot = plsc.cumsum(t)[15]
                sq = plsc.cumsum(q)[15]
                mean = tot * (1.0 / 64.0)
                var = sq * (1.0 / 64.0) - mean * mean
                inv = _rsqrt(var + EPS)
                mb = lax.broadcast_in_dim(mean, (16,), ())
                ib = lax.broadcast_in_dim(inv, (16,), ())
                for c in range(4):
                    sl = pl.ds(c * 16, 16)
                    n = (v[c] - mb) * ib
                    rows_v[row, sl] = n * gam_v[sl] + bet_v[sl]
            return c2
        lax.fori_loop(0, CHUNK // 16, do_grp, 0)
        pltpu.sync_copy(rows_v, out_ref.at[pl.ds(chunk_base, CHUNK)])
        return carry
    lax.fori_loop(0, N_CHUNKS, do_chunk, 0)


@jax.jit
def _emb(x, seg, tok_table, pos_table, seg_table, gamma, beta):
    mesh = plsc.VectorSubcoreMesh(core_axis_name="c", subcore_axis_name="s")
    f = pl.kernel(
        _body,
        out_type=jax.ShapeDtypeStruct((N, D), jnp.float32),
        mesh=mesh,
        compiler_params=pltpu.CompilerParams(
            needs_layout_passes=False, use_tc_tiling_on_sc=False),
        scratch_types=[
            pltpu.VMEM((CHUNK,), jnp.int32),
            pltpu.VMEM((CHUNK,), jnp.int32),
            pltpu.VMEM((CHUNK, D), jnp.float32),
            pltpu.VMEM((256, D), jnp.float32),
            pltpu.VMEM((128, D), jnp.float32),
            pltpu.VMEM((2, D), jnp.float32),
            pltpu.VMEM((D,), jnp.float32),
            pltpu.VMEM((D,), jnp.float32),
            pltpu.SemaphoreType.DMA,
        ],
    )
    return f(x, seg, tok_table, pos_table, seg_table, gamma, beta)


def kernel(x, seg, tok_table, pos_table, seg_table, gamma, beta):
    out = _emb(x.reshape(N), seg.reshape(N), tok_table, pos_table,
               seg_table, gamma, beta)
    return out.reshape(B, L, D)
